# Initial kernel scaffold; baseline (speedup 1.0000x reference)
import jax
import jax.numpy as jnp
from jax.experimental import pallas as pl


def kernel(pos, x, batch, params):
    raise NotImplementedError("write your pallas kernel here")


# speedup vs baseline: 15.1968x; 15.1968x over previous
import functools

import jax
import jax.numpy as jnp
from jax import lax
from jax.experimental import pallas as pl
from jax.experimental.pallas import tpu as pltpu
import jax.experimental.pallas.tpu_sc as plsc

N = 10000
B = 16
K = 16
NUM_CLASSES = 40
EPS = 1e-5

NP = 10240
T = 256
NT = NP // T
CCH = 128
WFAST = 1536
MASK_BATCH = 1e10
MASK_PAD = 3e37
KNOCK = 1e38
NEG = -1e38


def _bn_scale(g):
    return g / jnp.sqrt(1.0 + EPS)


def _edgeconv1_body(wmax, lo_ref, xfull_ref, xblk_ref, brow_ref, bcol_ref,
                    w1_ref, b1_ref, w2_ref, b2_ref, w3_ref, b3_ref,
                    out_ref, d2_ref, e_ref):
    t = pl.program_id(0)
    lo = lo_ref[t] * CCH
    xi = xblk_ref[:]
    win = xfull_ref[pl.ds(lo, wmax), :]
    sqw = jnp.sum(win * win, axis=1, keepdims=True)
    lhs = jnp.concatenate([-2.0 * xi, jnp.ones((T, 1), jnp.float32)], axis=1)
    rhs = jnp.concatenate([win, sqw], axis=1)
    d2 = lax.dot_general(lhs, rhs, (((1,), (1,)), ((), ())),
                         preferred_element_type=jnp.float32)
    d2 = d2 + jnp.sum(xi * xi, axis=1, keepdims=True)
    brow = brow_ref[:, 0:1]
    bcol = bcol_ref[0:1, pl.ds(lo, wmax)]
    d2 = d2 + jnp.where(brow != bcol, MASK_BATCH, 0.0)
    colid = lax.broadcasted_iota(jnp.int32, (T, wmax), 1) + lo
    d2 = jnp.where(colid >= N, d2 + MASK_PAD, d2)
    d2_ref[:] = d2

    def topk_body(k, carry):
        d2v = d2_ref[:]
        m = jnp.min(d2v, axis=1, keepdims=True)
        cand = jnp.where(d2v == m, colid, NP)
        jmin = jnp.min(cand, axis=1, keepdims=True)
        selb = colid == jmin
        d2_ref[:] = jnp.where(selb, KNOCK, d2v)
        sel = selb.astype(jnp.float32)
        xj = lax.dot_general(sel, win, (((1,), (0,)), ((), ())),
                             preferred_element_type=jnp.float32)
        e_ref[pl.ds(k * T, T), 0:8] = xi
        e_ref[pl.ds(k * T, T), 8:16] = xj - xi
        return carry

    lax.fori_loop(0, K, topk_body, 0)

    h = e_ref[:]
    for (w_r, b_r) in ((w1_ref, b1_ref), (w2_ref, b2_ref), (w3_ref, b3_ref)):
        h = jnp.dot(h, w_r[:], preferred_element_type=jnp.float32)
        h = jax.nn.relu(h + b_r[0:1, 0:64])
        h = h * b_r[1:2, 0:64] + b_r[2:3, 0:64]
    acc = h[0:T, :]
    for k in range(1, K):
        acc = jnp.maximum(acc, h[k * T:(k + 1) * T, :])
    out_ref[:] = jnp.concatenate(
        [acc, jnp.zeros((T, 64), jnp.float32)], axis=1)


def _make_edgeconv1(wmax):
    grid_spec = pltpu.PrefetchScalarGridSpec(
        num_scalar_prefetch=1,
        grid=(NT,),
        in_specs=[
            pl.BlockSpec((NP, 8), lambda t, s: (0, 0)),
            pl.BlockSpec((T, 8), lambda t, s: (t, 0)),
            pl.BlockSpec((T, 1), lambda t, s: (t, 0)),
            pl.BlockSpec((1, NP), lambda t, s: (0, 0)),
            pl.BlockSpec((16, 64), lambda t, s: (0, 0)),
            pl.BlockSpec((3, 64), lambda t, s: (0, 0)),
            pl.BlockSpec((64, 64), lambda t, s: (0, 0)),
            pl.BlockSpec((3, 64), lambda t, s: (0, 0)),
            pl.BlockSpec((64, 64), lambda t, s: (0, 0)),
            pl.BlockSpec((3, 64), lambda t, s: (0, 0)),
        ],
        out_specs=pl.BlockSpec((T, 128), lambda t, s: (t, 0)),
        scratch_shapes=[
            pltpu.VMEM((T, wmax), jnp.float32),
            pltpu.VMEM((K * T, 16), jnp.float32),
        ],
    )
    return pl.pallas_call(
        functools.partial(_edgeconv1_body, wmax),
        grid_spec=grid_spec,
        out_shape=jax.ShapeDtypeStruct((NP, 128), jnp.float32),
        compiler_params=pltpu.CompilerParams(
            dimension_semantics=("arbitrary",)),
    )


def _knn2_body(wmax, lo_ref, xfull_ref, xblk_ref, brow_ref, bcol_ref,
               idx_ref, d2_ref, idxs_ref):
    t = pl.program_id(0)
    lo = lo_ref[t] * CCH
    xi = xblk_ref[:, 0:64]
    win = xfull_ref[pl.ds(lo, wmax), 0:64]
    sqw = jnp.sum(win * win, axis=1, keepdims=True)
    lhs = jnp.concatenate([-2.0 * xi, jnp.ones((T, 1), jnp.float32)], axis=1)
    rhs = jnp.concatenate([win, sqw], axis=1)
    d2 = lax.dot_general(lhs, rhs, (((1,), (1,)), ((), ())),
                         preferred_element_type=jnp.float32)
    d2 = d2 + jnp.sum(xi * xi, axis=1, keepdims=True)
    brow = brow_ref[:, 0:1]
    bcol = bcol_ref[0:1, pl.ds(lo, wmax)]
    d2 = d2 + jnp.where(brow != bcol, MASK_BATCH, 0.0)
    colid = lax.broadcasted_iota(jnp.int32, (T, wmax), 1) + lo
    d2 = jnp.where(colid >= N, d2 + MASK_PAD, d2)
    d2_ref[:] = d2

    def topk_body(k, carry):
        d2v = d2_ref[:]
        m = jnp.min(d2v, axis=1, keepdims=True)
        cand = jnp.where(d2v == m, colid, NP)
        jmin = jnp.min(cand, axis=1, keepdims=True)
        d2_ref[:] = jnp.where(colid == jmin, KNOCK, d2v)
        idxs_ref[pl.ds(k * T, T), :] = jmin
        return carry

    lax.fori_loop(0, K, topk_body, 0)
    for k in range(K):
        idx_ref[:, k:k + 1] = idxs_ref[k * T:(k + 1) * T, :]


def _make_knn2(wmax):
    grid_spec = pltpu.PrefetchScalarGridSpec(
        num_scalar_prefetch=1,
        grid=(NT,),
        in_specs=[
            pl.BlockSpec((NP, 128), lambda t, s: (0, 0)),
            pl.BlockSpec((T, 128), lambda t, s: (t, 0)),
            pl.BlockSpec((T, 1), lambda t, s: (t, 0)),
            pl.BlockSpec((1, NP), lambda t, s: (0, 0)),
        ],
        out_specs=pl.BlockSpec((T, K), lambda t, s: (t, 0)),
        scratch_shapes=[pltpu.VMEM((T, wmax), jnp.float32),
                        pltpu.VMEM((K * T, 1), jnp.int32)],
    )
    return pl.pallas_call(
        functools.partial(_knn2_body, wmax),
        grid_spec=grid_spec,
        out_shape=jax.ShapeDtypeStruct((NP, K), jnp.int32),
        compiler_params=pltpu.CompilerParams(
            dimension_semantics=("arbitrary",)),
    )


_SC_CHUNK = 128


def _make_sc_gather():
    info = plsc.get_sparse_core_info()
    nw = info.num_cores * info.num_subcores
    epw = (K * NP) // nw
    nchunk = epw // _SC_CHUNK
    mesh = plsc.VectorSubcoreMesh(core_axis_name="c", subcore_axis_name="s")

    @functools.partial(
        pl.kernel,
        out_type=jax.ShapeDtypeStruct((K * NP, 128), jnp.float32),
        mesh=mesh,
        scratch_types=[
            pltpu.VMEM((nchunk, _SC_CHUNK), jnp.int32),
            pltpu.VMEM((_SC_CHUNK, 128), jnp.float32),
            pltpu.SemaphoreType.DMA,
        ],
    )
    def sc_gather(table_hbm, idx_hbm, out_hbm, idx_v, rows_v, sem):
        wid = lax.axis_index("s") * info.num_cores + lax.axis_index("c")
        pltpu.sync_copy(idx_hbm.at[wid], idx_v)

        def body(c, _):
            pltpu.async_copy(table_hbm.at[idx_v.at[c]], rows_v, sem).wait()
            row0 = pl.multiple_of(wid * epw + c * _SC_CHUNK, _SC_CHUNK)
            pltpu.sync_copy(rows_v, out_hbm.at[pl.ds(row0, _SC_CHUNK)])
            return 0

        lax.fori_loop(0, nchunk, body, 0)

    return sc_gather, nw, nchunk


def _tail_body(bmin_ref, bmax_ref, x1_ref, xj_ref, brow_ref,
               w2_ref, b2_ref, wl_ref, bl_ref,
               m1_ref, bm1_ref, m2_ref, bm2_ref, m3_ref, bm3_ref,
               out_ref, e_ref, seg_ref):
    t = pl.program_id(0)
    xi = x1_ref[:, 0:64]
    for k in range(K):
        xj = xj_ref[k][:, 0:64]
        e_ref[k * T:(k + 1) * T, 0:64] = xi
        e_ref[k * T:(k + 1) * T, 64:128] = xj - xi
    h = jnp.dot(e_ref[:], w2_ref[:], preferred_element_type=jnp.float32)
    h = jax.nn.relu(h + b2_ref[0:1, :])
    h = h * b2_ref[1:2, :] + b2_ref[2:3, :]
    x2 = h[0:T, :]
    for k in range(1, K):
        x2 = jnp.maximum(x2, h[k * T:(k + 1) * T, :])
    feat = jnp.concatenate([xi, x2], axis=1)
    o1 = jnp.dot(feat, wl_ref[:], preferred_element_type=jnp.float32)
    o1 = o1 + bl_ref[0:1, :]

    @pl.when(t == 0)
    def _init():
        seg_ref[:] = jnp.full((B, 1024), NEG, jnp.float32)

    brow = brow_ref[:, 0:1]

    def seg_body(b, carry):
        contrib = jnp.max(jnp.where(brow == b, o1, NEG), axis=0,
                          keepdims=True)
        seg_ref[pl.ds(b, 1), :] = jnp.maximum(seg_ref[pl.ds(b, 1), :],
                                              contrib)
        return carry

    lax.fori_loop(bmin_ref[t], bmax_ref[t] + 1, seg_body, 0)

    @pl.when(t == NT - 1)
    def _head():
        hh = seg_ref[:]
        hh = jax.nn.relu(jnp.dot(hh, m1_ref[:],
                                 preferred_element_type=jnp.float32)
                         + bm1_ref[0:1, :])
        hh = jax.nn.relu(jnp.dot(hh, m2_ref[:],
                                 preferred_element_type=jnp.float32)
                         + bm2_ref[0:1, :])
        hh = jnp.dot(hh, m3_ref[:], preferred_element_type=jnp.float32)
        hh = hh + bm3_ref[0:1, :]
        mx = jnp.max(hh, axis=1, keepdims=True)
        lse = jnp.log(jnp.sum(jnp.exp(hh - mx), axis=1, keepdims=True))
        out_ref[:] = hh - mx - lse


def _make_tail():
    grid_spec = pltpu.PrefetchScalarGridSpec(
        num_scalar_prefetch=2,
        grid=(NT,),
        in_specs=[
            pl.BlockSpec((T, 128), lambda t, s1, s2: (t, 0)),
            pl.BlockSpec((K, T, 128), lambda t, s1, s2: (0, t, 0)),
            pl.BlockSpec((T, 1), lambda t, s1, s2: (t, 0)),
            pl.BlockSpec((128, 128), lambda t, s1, s2: (0, 0)),
            pl.BlockSpec((3, 128), lambda t, s1, s2: (0, 0)),
            pl.BlockSpec((192, 1024), lambda t, s1, s2: (0, 0)),
            pl.BlockSpec((1, 1024), lambda t, s1, s2: (0, 0)),
            pl.BlockSpec((1024, 512), lambda t, s1, s2: (0, 0)),
            pl.BlockSpec((1, 512), lambda t, s1, s2: (0, 0)),
            pl.BlockSpec((512, 256), lambda t, s1, s2: (0, 0)),
            pl.BlockSpec((1, 256), lambda t, s1, s2: (0, 0)),
            pl.BlockSpec((256, 128), lambda t, s1, s2: (0, 0)),
            pl.BlockSpec((1, 128), lambda t, s1, s2: (0, 0)),
        ],
        out_specs=pl.BlockSpec((B, 128), lambda t, s1, s2: (0, 0)),
        scratch_shapes=[
            pltpu.VMEM((K * T, 128), jnp.float32),
            pltpu.VMEM((B, 1024), jnp.float32),
        ],
    )
    return pl.pallas_call(
        _tail_body,
        grid_spec=grid_spec,
        out_shape=jax.ShapeDtypeStruct((B, 128), jnp.float32),
        compiler_params=pltpu.CompilerParams(
            dimension_semantics=("arbitrary",)),
    )


def _stack_bn(b, g, beta):
    s = _bn_scale(g)
    return jnp.stack([b, s, beta], axis=0)


def kernel(pos, x, batch, params):
    batch = batch.astype(jnp.int32)
    x0 = jnp.concatenate([pos, x], axis=1)
    x0p = jnp.zeros((NP, 8), jnp.float32).at[:N, :4].set(x0)
    bpad = jnp.full((NP,), -1, jnp.int32).at[:N].set(batch)
    brow = bpad.reshape(NP, 1)
    bcol = bpad.reshape(1, NP)

    arangeb = jnp.arange(B, dtype=jnp.int32)
    starts = jnp.searchsorted(batch, arangeb, side="left").astype(jnp.int32)
    ends = jnp.searchsorted(batch, arangeb, side="right").astype(jnp.int32)
    counts = ends - starts
    small_seg = jnp.any((counts > 0) & (counts < K))

    tidx = jnp.arange(NT, dtype=jnp.int32)
    r0 = jnp.minimum(tidx * T, N - 1)
    r1 = jnp.minimum(tidx * T + (T - 1), N - 1)
    bmin = batch[r0]
    bmax = batch[r1]
    lo = starts[bmin]
    hi = ends[bmax]
    nch_fast = WFAST // CCH
    lo_chunk = jnp.minimum(lo // CCH, NP // CCH - nch_fast)
    fits = jnp.all(hi <= (lo_chunk + nch_fast) * CCH)
    fast_ok = fits & jnp.logical_not(small_seg)

    (w1, b1, g1, be1), (w2, b2, g2, be2), (w3, b3, g3, be3) = params["conv1"]
    w1p = jnp.zeros((16, 64), jnp.float32)
    w1p = w1p.at[0:4, :].set(w1[:, 0:4].T)
    w1p = w1p.at[8:12, :].set(w1[:, 4:8].T)
    bn1 = _stack_bn(b1, g1, be1)
    bn2 = _stack_bn(b2, g2, be2)
    bn3 = _stack_bn(b3, g3, be3)
    (wc2, bc2, gc2, bec2), = params["conv2"]
    bnc2 = _stack_bn(bc2, gc2, bec2)
    wlin, blin = params["lin1"]
    (wm1, bm1), (wm2, bm2), (wm3, bm3) = params["mlp"]
    wm3p = jnp.zeros((256, 128), jnp.float32).at[:, :NUM_CLASSES].set(wm3.T)
    bm3p = jnp.full((1, 128), -1e30, jnp.float32).at[0, :NUM_CLASSES].set(bm3)

    conv1_w = (w1p, bn1, w2.T, bn2, w3.T, bn3)

    def knn_phase(wm, lo_arr, ops):
        (x0p_, brow_, bcol_, cw) = ops
        x1 = _make_edgeconv1(wm)(lo_arr, x0p_, x0p_, brow_, bcol_, *cw)
        idx = _make_knn2(wm)(lo_arr, x1, x1, brow_, bcol_)
        return x1, idx

    ops = (x0p, brow, bcol, conv1_w)
    zeros_lo = jnp.zeros((NT,), jnp.int32)
    x1, idx = lax.cond(
        fast_ok,
        lambda o: knn_phase(WFAST, lo_chunk, o),
        lambda o: knn_phase(NP, zeros_lo, o),
        ops)

    sc_gather, nw, nchunk = _make_sc_gather()
    idx_km = jnp.zeros((K, NP), jnp.int32).at[:, :N].set(idx[:N, :].T)
    idx_w = idx_km.reshape(nw, nchunk, _SC_CHUNK)
    xj = sc_gather(x1, idx_w)
    xj = xj.reshape(K, NP, 128)

    out = _make_tail()(bmin, bmax, x1, xj, brow,
                       wc2.T, bnc2, wlin.T, blin.reshape(1, 1024),
                       wm1.T, bm1.reshape(1, 512),
                       wm2.T, bm2.reshape(1, 256),
                       wm3p, bm3p)
    return out[:, :NUM_CLASSES]


# speedup vs baseline: 15.3148x; 1.0078x over previous
import functools

import jax
import jax.numpy as jnp
from jax import lax
from jax.experimental import pallas as pl
from jax.experimental.pallas import tpu as pltpu
import jax.experimental.pallas.tpu_sc as plsc

N = 10000
B = 16
K = 16
NUM_CLASSES = 40
EPS = 1e-5

NP = 10240
T = 256
NT = NP // T
CCH = 128
WFAST = 1536
MASK_BATCH = 1e10
MASK_PAD = 3e37
KNOCK = 1e38
NEG = -1e38


def _bn_scale(g):
    return g / jnp.sqrt(1.0 + EPS)


def _edgeconv1_body(wmax, lo_ref, xfull_ref, xblk_ref, brow_ref, bcol_ref,
                    w1_ref, b1_ref, w2_ref, b2_ref, w3_ref, b3_ref,
                    out_ref, d2_ref, e_ref):
    t = pl.program_id(0)
    lo = lo_ref[t] * CCH
    xi = xblk_ref[:]
    win = xfull_ref[pl.ds(lo, wmax), :]
    sqw = jnp.sum(win * win, axis=1, keepdims=True)
    lhs = jnp.concatenate([-2.0 * xi, jnp.ones((T, 1), jnp.float32)], axis=1)
    rhs = jnp.concatenate([win, sqw], axis=1)
    d2 = lax.dot_general(lhs, rhs, (((1,), (1,)), ((), ())),
                         preferred_element_type=jnp.float32)
    d2 = d2 + jnp.sum(xi * xi, axis=1, keepdims=True)
    brow = brow_ref[:, 0:1]
    bcol = bcol_ref[0:1, pl.ds(lo, wmax)]
    d2 = d2 + jnp.where(brow != bcol, MASK_BATCH, 0.0)
    colid = lax.broadcasted_iota(jnp.int32, (T, wmax), 1) + lo
    d2 = jnp.where(colid >= N, d2 + MASK_PAD, d2)
    d2_ref[:] = d2

    def topk_body(k, carry):
        d2v = d2_ref[:]
        m = jnp.min(d2v, axis=1, keepdims=True)
        cand = jnp.where(d2v == m, colid, NP)
        jmin = jnp.min(cand, axis=1, keepdims=True)
        selb = colid == jmin
        d2_ref[:] = jnp.where(selb, KNOCK, d2v)
        sel = selb.astype(jnp.float32)
        xj = lax.dot_general(sel, win, (((1,), (0,)), ((), ())),
                             preferred_element_type=jnp.float32)
        e_ref[pl.ds(k * T, T), 0:8] = xi
        e_ref[pl.ds(k * T, T), 8:16] = xj - xi
        return carry

    lax.fori_loop(0, K, topk_body, 0)

    h = e_ref[:]
    for (w_r, b_r) in ((w1_ref, b1_ref), (w2_ref, b2_ref), (w3_ref, b3_ref)):
        h = jnp.dot(h, w_r[:], preferred_element_type=jnp.float32)
        h = jax.nn.relu(h + b_r[0:1, 0:64])
        h = h * b_r[1:2, 0:64] + b_r[2:3, 0:64]
    acc = h[0:T, :]
    for k in range(1, K):
        acc = jnp.maximum(acc, h[k * T:(k + 1) * T, :])
    out_ref[:] = jnp.concatenate(
        [acc, jnp.zeros((T, 64), jnp.float32)], axis=1)


def _make_edgeconv1(wmax):
    grid_spec = pltpu.PrefetchScalarGridSpec(
        num_scalar_prefetch=1,
        grid=(NT,),
        in_specs=[
            pl.BlockSpec((NP, 8), lambda t, s: (0, 0)),
            pl.BlockSpec((T, 8), lambda t, s: (t, 0)),
            pl.BlockSpec((T, 1), lambda t, s: (t, 0)),
            pl.BlockSpec((1, NP), lambda t, s: (0, 0)),
            pl.BlockSpec((16, 64), lambda t, s: (0, 0)),
            pl.BlockSpec((3, 64), lambda t, s: (0, 0)),
            pl.BlockSpec((64, 64), lambda t, s: (0, 0)),
            pl.BlockSpec((3, 64), lambda t, s: (0, 0)),
            pl.BlockSpec((64, 64), lambda t, s: (0, 0)),
            pl.BlockSpec((3, 64), lambda t, s: (0, 0)),
        ],
        out_specs=pl.BlockSpec((T, 128), lambda t, s: (t, 0)),
        scratch_shapes=[
            pltpu.VMEM((T, wmax), jnp.float32),
            pltpu.VMEM((K * T, 16), jnp.float32),
        ],
    )
    return pl.pallas_call(
        functools.partial(_edgeconv1_body, wmax),
        grid_spec=grid_spec,
        out_shape=jax.ShapeDtypeStruct((NP, 128), jnp.float32),
        compiler_params=pltpu.CompilerParams(
            dimension_semantics=("arbitrary",)),
    )


def _knn2_body(wmax, lo_ref, xfull_ref, xblk_ref, brow_ref, bcol_ref,
               idx_ref, d2_ref, idxs_ref):
    t = pl.program_id(0)
    lo = lo_ref[t] * CCH
    xi = xblk_ref[:, 0:64]
    win = xfull_ref[pl.ds(lo, wmax), 0:64]
    sqw = jnp.sum(win * win, axis=1, keepdims=True)
    lhs = jnp.concatenate([-2.0 * xi, jnp.ones((T, 1), jnp.float32)], axis=1)
    rhs = jnp.concatenate([win, sqw], axis=1)
    d2 = lax.dot_general(lhs, rhs, (((1,), (1,)), ((), ())),
                         preferred_element_type=jnp.float32)
    d2 = d2 + jnp.sum(xi * xi, axis=1, keepdims=True)
    brow = brow_ref[:, 0:1]
    bcol = bcol_ref[0:1, pl.ds(lo, wmax)]
    d2 = d2 + jnp.where(brow != bcol, MASK_BATCH, 0.0)
    colid = lax.broadcasted_iota(jnp.int32, (T, wmax), 1) + lo
    d2 = jnp.where(colid >= N, d2 + MASK_PAD, d2)
    d2_ref[:] = d2

    def topk_body(k, carry):
        d2v = d2_ref[:]
        m = jnp.min(d2v, axis=1, keepdims=True)
        cand = jnp.where(d2v == m, colid, NP)
        jmin = jnp.min(cand, axis=1, keepdims=True)
        d2_ref[:] = jnp.where(colid == jmin, KNOCK, d2v)
        idxs_ref[pl.ds(k * T, T), :] = jmin
        return carry

    lax.fori_loop(0, K, topk_body, 0)
    for k in range(K):
        idx_ref[:, k:k + 1] = idxs_ref[k * T:(k + 1) * T, :]


def _make_knn2(wmax):
    grid_spec = pltpu.PrefetchScalarGridSpec(
        num_scalar_prefetch=1,
        grid=(NT,),
        in_specs=[
            pl.BlockSpec((NP, 128), lambda t, s: (0, 0)),
            pl.BlockSpec((T, 128), lambda t, s: (t, 0)),
            pl.BlockSpec((T, 1), lambda t, s: (t, 0)),
            pl.BlockSpec((1, NP), lambda t, s: (0, 0)),
        ],
        out_specs=pl.BlockSpec((T, K), lambda t, s: (t, 0)),
        scratch_shapes=[pltpu.VMEM((T, wmax), jnp.float32),
                        pltpu.VMEM((K * T, 1), jnp.int32)],
    )
    return pl.pallas_call(
        functools.partial(_knn2_body, wmax),
        grid_spec=grid_spec,
        out_shape=jax.ShapeDtypeStruct((NP, K), jnp.int32),
        compiler_params=pltpu.CompilerParams(
            dimension_semantics=("arbitrary",)),
    )


_SC_CHUNK = 128


def _make_sc_gather():
    info = plsc.get_sparse_core_info()
    nw = info.num_cores * info.num_subcores
    epw = (K * NP) // nw
    nchunk = epw // _SC_CHUNK
    mesh = plsc.VectorSubcoreMesh(core_axis_name="c", subcore_axis_name="s")

    @functools.partial(
        pl.kernel,
        out_type=jax.ShapeDtypeStruct((K * NP, 128), jnp.float32),
        mesh=mesh,
        scratch_types=[
            pltpu.VMEM((nchunk, _SC_CHUNK), jnp.int32),
            pltpu.VMEM((2, _SC_CHUNK, 128), jnp.float32),
            pltpu.SemaphoreType.DMA,
            pltpu.SemaphoreType.DMA,
        ],
    )
    def sc_gather(table_hbm, idx_hbm, out_hbm, idx_v, rows_v, gsem, ssem):
        wid = lax.axis_index("s") * info.num_cores + lax.axis_index("c")
        pltpu.sync_copy(idx_hbm.at[wid], idx_v)
        base = wid * epw

        def out_at(c):
            row0 = pl.multiple_of(base + c * _SC_CHUNK, _SC_CHUNK)
            return out_hbm.at[pl.ds(row0, _SC_CHUNK)]

        def gather_start(c):
            pltpu.make_async_copy(
                table_hbm.at[idx_v.at[c]], rows_v.at[c % 2], gsem).start()

        def gather_wait(c):
            pltpu.make_async_copy(
                table_hbm.at[idx_v.at[c]], rows_v.at[c % 2], gsem).wait()

        def scatter_start(c):
            pltpu.make_async_copy(rows_v.at[c % 2], out_at(c), ssem).start()

        def scatter_wait(c):
            pltpu.make_async_copy(rows_v.at[c % 2], out_at(c), ssem).wait()

        gather_start(0)

        def body(c, _):
            gather_wait(c)
            scatter_start(c)

            @pl.when(c >= 1)
            def _drain():
                scatter_wait(c - 1)

            @pl.when(c + 1 < nchunk)
            def _next():
                gather_start(c + 1)

            return 0

        lax.fori_loop(0, nchunk, body, 0)
        scatter_wait(nchunk - 1)

    return sc_gather, nw, nchunk


def _tail_body(bmin_ref, bmax_ref, x1_ref, xj_ref, brow_ref,
               w2_ref, b2_ref, wl_ref, bl_ref,
               m1_ref, bm1_ref, m2_ref, bm2_ref, m3_ref, bm3_ref,
               out_ref, e_ref, seg_ref):
    t = pl.program_id(0)
    xi = x1_ref[:, 0:64]
    for k in range(K):
        xj = xj_ref[k][:, 0:64]
        e_ref[k * T:(k + 1) * T, 0:64] = xi
        e_ref[k * T:(k + 1) * T, 64:128] = xj - xi
    h = jnp.dot(e_ref[:], w2_ref[:], preferred_element_type=jnp.float32)
    h = jax.nn.relu(h + b2_ref[0:1, :])
    h = h * b2_ref[1:2, :] + b2_ref[2:3, :]
    x2 = h[0:T, :]
    for k in range(1, K):
        x2 = jnp.maximum(x2, h[k * T:(k + 1) * T, :])
    feat = jnp.concatenate([xi, x2], axis=1)
    o1 = jnp.dot(feat, wl_ref[:], preferred_element_type=jnp.float32)
    o1 = o1 + bl_ref[0:1, :]

    @pl.when(t == 0)
    def _init():
        seg_ref[:] = jnp.full((B, 1024), NEG, jnp.float32)

    brow = brow_ref[:, 0:1]

    def seg_body(b, carry):
        contrib = jnp.max(jnp.where(brow == b, o1, NEG), axis=0,
                          keepdims=True)
        seg_ref[pl.ds(b, 1), :] = jnp.maximum(seg_ref[pl.ds(b, 1), :],
                                              contrib)
        return carry

    lax.fori_loop(bmin_ref[t], bmax_ref[t] + 1, seg_body, 0)

    @pl.when(t == NT - 1)
    def _head():
        hh = seg_ref[:]
        hh = jax.nn.relu(jnp.dot(hh, m1_ref[:],
                                 preferred_element_type=jnp.float32)
                         + bm1_ref[0:1, :])
        hh = jax.nn.relu(jnp.dot(hh, m2_ref[:],
                                 preferred_element_type=jnp.float32)
                         + bm2_ref[0:1, :])
        hh = jnp.dot(hh, m3_ref[:], preferred_element_type=jnp.float32)
        hh = hh + bm3_ref[0:1, :]
        mx = jnp.max(hh, axis=1, keepdims=True)
        lse = jnp.log(jnp.sum(jnp.exp(hh - mx), axis=1, keepdims=True))
        out_ref[:] = hh - mx - lse


def _make_tail():
    grid_spec = pltpu.PrefetchScalarGridSpec(
        num_scalar_prefetch=2,
        grid=(NT,),
        in_specs=[
            pl.BlockSpec((T, 128), lambda t, s1, s2: (t, 0)),
            pl.BlockSpec((K, T, 128), lambda t, s1, s2: (0, t, 0)),
            pl.BlockSpec((T, 1), lambda t, s1, s2: (t, 0)),
            pl.BlockSpec((128, 128), lambda t, s1, s2: (0, 0)),
            pl.BlockSpec((3, 128), lambda t, s1, s2: (0, 0)),
            pl.BlockSpec((192, 1024), lambda t, s1, s2: (0, 0)),
            pl.BlockSpec((1, 1024), lambda t, s1, s2: (0, 0)),
            pl.BlockSpec((1024, 512), lambda t, s1, s2: (0, 0)),
            pl.BlockSpec((1, 512), lambda t, s1, s2: (0, 0)),
            pl.BlockSpec((512, 256), lambda t, s1, s2: (0, 0)),
            pl.BlockSpec((1, 256), lambda t, s1, s2: (0, 0)),
            pl.BlockSpec((256, 128), lambda t, s1, s2: (0, 0)),
            pl.BlockSpec((1, 128), lambda t, s1, s2: (0, 0)),
        ],
        out_specs=pl.BlockSpec((B, 128), lambda t, s1, s2: (0, 0)),
        scratch_shapes=[
            pltpu.VMEM((K * T, 128), jnp.float32),
            pltpu.VMEM((B, 1024), jnp.float32),
        ],
    )
    return pl.pallas_call(
        _tail_body,
        grid_spec=grid_spec,
        out_shape=jax.ShapeDtypeStruct((B, 128), jnp.float32),
        compiler_params=pltpu.CompilerParams(
            dimension_semantics=("arbitrary",)),
    )


def _stack_bn(b, g, beta):
    s = _bn_scale(g)
    return jnp.stack([b, s, beta], axis=0)


def kernel(pos, x, batch, params):
    batch = batch.astype(jnp.int32)
    x0 = jnp.concatenate([pos, x], axis=1)
    x0p = jnp.zeros((NP, 8), jnp.float32).at[:N, :4].set(x0)
    bpad = jnp.full((NP,), -1, jnp.int32).at[:N].set(batch)
    brow = bpad.reshape(NP, 1)
    bcol = bpad.reshape(1, NP)

    arangeb = jnp.arange(B, dtype=jnp.int32)
    starts = jnp.searchsorted(batch, arangeb, side="left").astype(jnp.int32)
    ends = jnp.searchsorted(batch, arangeb, side="right").astype(jnp.int32)
    counts = ends - starts
    small_seg = jnp.any((counts > 0) & (counts < K))

    tidx = jnp.arange(NT, dtype=jnp.int32)
    r0 = jnp.minimum(tidx * T, N - 1)
    r1 = jnp.minimum(tidx * T + (T - 1), N - 1)
    bmin = batch[r0]
    bmax = batch[r1]
    lo = starts[bmin]
    hi = ends[bmax]
    nch_fast = WFAST // CCH
    lo_chunk = jnp.minimum(lo // CCH, NP // CCH - nch_fast)
    fits = jnp.all(hi <= (lo_chunk + nch_fast) * CCH)
    fast_ok = fits & jnp.logical_not(small_seg)

    (w1, b1, g1, be1), (w2, b2, g2, be2), (w3, b3, g3, be3) = params["conv1"]
    w1p = jnp.zeros((16, 64), jnp.float32)
    w1p = w1p.at[0:4, :].set(w1[:, 0:4].T)
    w1p = w1p.at[8:12, :].set(w1[:, 4:8].T)
    bn1 = _stack_bn(b1, g1, be1)
    bn2 = _stack_bn(b2, g2, be2)
    bn3 = _stack_bn(b3, g3, be3)
    (wc2, bc2, gc2, bec2), = params["conv2"]
    bnc2 = _stack_bn(bc2, gc2, bec2)
    wlin, blin = params["lin1"]
    (wm1, bm1), (wm2, bm2), (wm3, bm3) = params["mlp"]
    wm3p = jnp.zeros((256, 128), jnp.float32).at[:, :NUM_CLASSES].set(wm3.T)
    bm3p = jnp.full((1, 128), -1e30, jnp.float32).at[0, :NUM_CLASSES].set(bm3)

    conv1_w = (w1p, bn1, w2.T, bn2, w3.T, bn3)

    def knn_phase(wm, lo_arr, ops):
        (x0p_, brow_, bcol_, cw) = ops
        x1 = _make_edgeconv1(wm)(lo_arr, x0p_, x0p_, brow_, bcol_, *cw)
        idx = _make_knn2(wm)(lo_arr, x1, x1, brow_, bcol_)
        return x1, idx

    ops = (x0p, brow, bcol, conv1_w)
    zeros_lo = jnp.zeros((NT,), jnp.int32)
    x1, idx = lax.cond(
        fast_ok,
        lambda o: knn_phase(WFAST, lo_chunk, o),
        lambda o: knn_phase(NP, zeros_lo, o),
        ops)

    sc_gather, nw, nchunk = _make_sc_gather()
    idx_km = jnp.zeros((K, NP), jnp.int32).at[:, :N].set(idx[:N, :].T)
    idx_w = idx_km.reshape(nw, nchunk, _SC_CHUNK)
    xj = sc_gather(x1, idx_w)
    xj = xj.reshape(K, NP, 128)

    out = _make_tail()(bmin, bmax, x1, xj, brow,
                       wc2.T, bnc2, wlin.T, blin.reshape(1, 1024),
                       wm1.T, bm1.reshape(1, 512),
                       wm2.T, bm2.reshape(1, 256),
                       wm3p, bm3p)
    return out[:, :NUM_CLASSES]
